# baseline (speedup 1.0000x reference)
import functools

import jax
import jax.numpy as jnp
from jax import lax
from jax.experimental import pallas as pl
from jax.experimental.pallas import tpu as pltpu
from jax.experimental.pallas import tpu_sc as plsc

N_CORES = 2
N_SUBCORES = 16
NW = N_CORES * N_SUBCORES
CHUNK = 128


def _sc_gather(user_table, item_table, users2d, items2d, batch, dim):
    b_per_w = batch // NW
    n_chunks = b_per_w // CHUNK
    mesh = plsc.VectorSubcoreMesh(core_axis_name="c", subcore_axis_name="s")

    @functools.partial(
        pl.kernel,
        mesh=mesh,
        out_type=[
            jax.ShapeDtypeStruct((batch, dim), jnp.float32),
            jax.ShapeDtypeStruct((batch, dim), jnp.float32),
        ],
        scratch_types=[
            pltpu.VMEM((n_chunks, CHUNK), jnp.int32),
            pltpu.VMEM((n_chunks, CHUNK), jnp.int32),
            pltpu.VMEM((b_per_w, dim), jnp.float32),
            pltpu.VMEM((b_per_w, dim), jnp.float32),
            pltpu.SemaphoreType.DMA,
        ],
        compiler_params=pltpu.CompilerParams(use_tc_tiling_on_sc=False),
    )
    def gather_kernel(u_tab, i_tab, u_idx_hbm, i_idx_hbm, u_out, i_out,
                      u_idx, i_idx, u_rows, i_rows, sem):
        wid = lax.axis_index("s") * N_CORES + lax.axis_index("c")
        base = wid * b_per_w
        row0 = wid * n_chunks
        pltpu.sync_copy(u_idx_hbm.at[pl.ds(row0, n_chunks)], u_idx)
        pltpu.sync_copy(i_idx_hbm.at[pl.ds(row0, n_chunks)], i_idx)
        copies = []
        for c in range(n_chunks):
            copies.append(pltpu.async_copy(
                u_tab.at[u_idx.at[c]], u_rows.at[pl.ds(c * CHUNK, CHUNK)], sem))
            copies.append(pltpu.async_copy(
                i_tab.at[i_idx.at[c]], i_rows.at[pl.ds(c * CHUNK, CHUNK)], sem))
        for cp in copies:
            cp.wait()
        pltpu.sync_copy(u_rows, u_out.at[pl.ds(base, b_per_w)])
        pltpu.sync_copy(i_rows, i_out.at[pl.ds(base, b_per_w)])

    return gather_kernel(user_table, item_table, users2d, items2d)


def _score_body(u_ref, i_ref, o_ref):
    o_ref[...] = jnp.sum(u_ref[...] * i_ref[...], axis=1, keepdims=True)


def kernel(users, items, user_table, item_table):
    batch = users.shape[0]
    dim = user_table.shape[1]
    users2d = users.astype(jnp.int32).reshape(batch // CHUNK, CHUNK)
    items2d = items.astype(jnp.int32).reshape(batch // CHUNK, CHUNK)
    user_emb, item_emb = _sc_gather(
        user_table, item_table, users2d, items2d, batch, dim)
    scores2d = pl.pallas_call(
        _score_body,
        out_shape=jax.ShapeDtypeStruct((batch, 1), jnp.float32),
    )(user_emb, item_emb)
    return user_emb, item_emb, scores2d.reshape(batch)
